# baseline (speedup 1.0000x reference)
import functools
import math

import jax
import jax.numpy as jnp
import numpy as np
from jax.experimental import pallas as pl
from jax.experimental.pallas import tpu as pltpu

B, T, DIN, DH, DM, K = 16, 4096, 32, 64, 128, 32
SEG = T // K
PACK = 4
TP = T // PACK
SEGP = SEG // PACK
DIN4 = DIN * PACK
DH4 = DH * PACK


def _fused_kernel(x_ref, w1_ref, b1_ref, w2_ref, b2_ref, g_ref, out_ref):
    x = x_ref[0]
    h = jnp.dot(x, w1_ref[...], preferred_element_type=jnp.float32) + b1_ref[...]
    h = 0.5 * h * (1.0 + jax.lax.erf(h * (1.0 / math.sqrt(2.0))))
    pooled = h.reshape(K, SEGP, DH4).sum(axis=1) * (1.0 / SEG)
    out = jnp.dot(pooled, w2_ref[...], preferred_element_type=jnp.float32)
    g = jnp.tanh(g_ref[0, 0])
    out_ref[0] = g * (out + b2_ref[...])


@jax.jit
def kernel(imu_seq, W1, b1, W2, b2, gate):
    x4 = imu_seq.reshape(B, TP, DIN4)
    W1b = jnp.kron(jnp.eye(PACK, dtype=W1.dtype), W1)
    b1t = jnp.tile(b1, PACK).reshape(1, DH4)
    W2s = jnp.concatenate([W2] * PACK, axis=0)
    grid = (B,)
    out = pl.pallas_call(
        _fused_kernel,
        grid=grid,
        in_specs=[
            pl.BlockSpec((1, TP, DIN4), lambda b: (b, 0, 0)),
            pl.BlockSpec((DIN4, DH4), lambda b: (0, 0)),
            pl.BlockSpec((1, DH4), lambda b: (0, 0)),
            pl.BlockSpec((DH4, DM), lambda b: (0, 0)),
            pl.BlockSpec((1, DM), lambda b: (0, 0)),
            pl.BlockSpec((1, 1), lambda b: (0, 0)),
        ],
        out_specs=pl.BlockSpec((1, K, DM), lambda b: (b, 0, 0)),
        out_shape=jax.ShapeDtypeStruct((B, K, DM), jnp.float32),
        compiler_params=pltpu.CompilerParams(
            dimension_semantics=("parallel",),
        ),
    )(
        x4,
        W1b,
        b1t,
        W2s,
        b2.reshape(1, DM),
        gate.reshape(1, 1),
    )
    return out


# speedup vs baseline: 1.6262x; 1.6262x over previous
import functools
import math

import jax
import jax.numpy as jnp
from jax.experimental import pallas as pl
from jax.experimental.pallas import tpu as pltpu

B, T, DIN, DH, DM, K = 16, 4096, 32, 64, 128, 32
SEG = T // K
GB = 2
NSTEP = B // (2 * GB)


def _mlp_pool(x, w1, b1, w2):
    x2 = x.reshape(GB * T, DIN)
    h = jnp.dot(x2, w1, preferred_element_type=jnp.float32) + b1
    h = 0.5 * h * (1.0 + jax.lax.erf(h * (1.0 / math.sqrt(2.0))))
    pooled = h.reshape(GB * K, SEG, DH).sum(axis=1) * (1.0 / SEG)
    return jnp.dot(pooled, w2, preferred_element_type=jnp.float32)


def _fused_kernel(x0_ref, x1_ref, w1_ref, b1_ref, w2_ref, b2_ref, g_ref,
                  o0_ref, o1_ref):
    w1 = w1_ref[...]
    b1 = b1_ref[...]
    w2 = w2_ref[...]
    g = jnp.tanh(g_ref[0, 0])
    out0 = _mlp_pool(x0_ref[...], w1, b1, w2)
    o0_ref[...] = (g * (out0 + b2_ref[...])).reshape(GB, K, DM)
    out1 = _mlp_pool(x1_ref[...], w1, b1, w2)
    o1_ref[...] = (g * (out1 + b2_ref[...])).reshape(GB, K, DM)


@jax.jit
def kernel(imu_seq, W1, b1, W2, b2, gate):
    outs = pl.pallas_call(
        _fused_kernel,
        grid=(NSTEP,),
        in_specs=[
            pl.BlockSpec((GB, T, DIN), lambda j: (j, 0, 0)),
            pl.BlockSpec((GB, T, DIN), lambda j: (j + NSTEP, 0, 0)),
            pl.BlockSpec((DIN, DH), lambda j: (0, 0)),
            pl.BlockSpec((1, DH), lambda j: (0, 0)),
            pl.BlockSpec((DH, DM), lambda j: (0, 0)),
            pl.BlockSpec((1, DM), lambda j: (0, 0)),
            pl.BlockSpec((1, 1), lambda j: (0, 0)),
        ],
        out_specs=[
            pl.BlockSpec((GB, K, DM), lambda j: (j, 0, 0)),
            pl.BlockSpec((GB, K, DM), lambda j: (j, 0, 0)),
        ],
        out_shape=[
            jax.ShapeDtypeStruct((B // 2, K, DM), jnp.float32),
            jax.ShapeDtypeStruct((B // 2, K, DM), jnp.float32),
        ],
        compiler_params=pltpu.CompilerParams(
            dimension_semantics=("arbitrary",),
        ),
    )(
        imu_seq,
        imu_seq,
        W1,
        b1.reshape(1, DH),
        W2,
        b2.reshape(1, DM),
        gate.reshape(1, 1),
    )
    return jnp.concatenate(outs, axis=0)
